# baseline (speedup 1.0000x reference)
import functools

import jax
import jax.numpy as jnp
from jax import lax
from jax.experimental import pallas as pl
from jax.experimental.pallas import tpu as pltpu
from jax.experimental.pallas import tpu_sc as plsc

N = 10000
E = 320000

NC = 2
NS = 16
FW = 64
C = 64
NCH = 320
E_PAD = NS * NCH * C
SLAB = 632
ACC_ROWS = NS * SLAB
KBITS = 14


def _splat_lane(v16, i):
    return lax.gather(
        v16,
        jnp.full((16, 1), i, jnp.int32),
        lax.GatherDimensionNumbers(
            offset_dims=(), collapsed_slice_dims=(0,), start_index_map=(0,)),
        (1,),
        mode=lax.GatherScatterMode.PROMISE_IN_BOUNDS,
    )


NBUF = 4


def _sc_agg_body(h_hbm, key_hbm, w_hbm,
                 out_hbm,
                 key_v, w_v, rbuf0, rbuf1, rbuf2, rbuf3,
                 sbuf0, sbuf1, sbuf2, sbuf3, src_sc, dst_sc,
                 acc_sh, gsem0, gsem1, gsem2, gsem3,
                 ssem0, ssem1, ssem2, ssem3):
    cid = lax.axis_index("c")
    sid = lax.axis_index("s")

    pltpu.sync_copy(key_hbm.at[sid], key_v)
    pltpu.sync_copy(w_hbm.at[sid], w_v)

    rbufs = (rbuf0, rbuf1, rbuf2, rbuf3)
    sbufs = (sbuf0, sbuf1, sbuf2, sbuf3)
    gsems = (gsem0, gsem1, gsem2, gsem3)
    ssems = (ssem0, ssem1, ssem2, ssem3)

    shift16 = jnp.full((16,), KBITS, jnp.int32)
    mask16 = jnp.full((16,), (1 << KBITS) - 1, jnp.int32)

    def unpack_src(row, b):
        for g in range(C // 16):
            sl = pl.ds(g * 16, 16)
            src_sc[b, sl] = lax.shift_right_logical(key_v[row, sl], shift16)

    def unpack_dst(row, b):
        for g in range(C // 16):
            sl = pl.ds(g * 16, 16)
            dst_sc[b, sl] = lax.bitwise_and(key_v[row, sl], mask16)

    zero16 = jnp.zeros((16,), jnp.float32)
    for r in range(C):
        for k in range(FW // 16):
            for sb in sbufs:
                sb[r, pl.ds(k * 16, 16)] = zero16
    base = sid * SLAB
    for j in range(SLAB // C):
        pltpu.sync_copy(sbuf0, acc_sh.at[pl.ds(base + j * C, C)])
    pltpu.sync_copy(sbuf0.at[pl.ds(0, SLAB % C)],
                    acc_sh.at[pl.ds(base + (SLAB // C) * C, SLAB % C)])

    plsc.subcore_barrier()

    htab = h_hbm

    def scatter_start(b):
        pltpu.async_copy(sbufs[b], acc_sh.at[dst_sc.at[b]], ssems[b], add=True)

    def scatter_wait(b):
        pltpu.make_async_copy(
            sbufs[b], acc_sh.at[dst_sc.at[b]], ssems[b]).wait()

    for b in range(NBUF):
        unpack_src(b, b)
        pltpu.async_copy(htab.at[src_sc.at[b]], rbufs[b], gsems[b])
        unpack_dst(b, b)

    def chunk_group(t, _):
        c0 = t * NBUF
        for b in range(NBUF):
            cur = c0 + b
            rbuf = rbufs[b]
            sbuf = sbufs[b]
            pltpu.make_async_copy(htab.at[src_sc.at[b]], rbuf, gsems[b]).wait()

            unpack_dst(cur, b)

            pass

            pass

            nxt = cur + NBUF

            @pl.when(nxt < NCH)
            def _():
                unpack_src(nxt, b)
                pltpu.async_copy(htab.at[src_sc.at[b]], rbuf, gsems[b])

        return ()

    lax.fori_loop(0, NCH // NBUF, chunk_group, (), unroll=False)

    pass

    plsc.subcore_barrier()

    pltpu.sync_copy(acc_sh.at[pl.ds(base, SLAB)],
                    out_hbm.at[cid].at[pl.ds(base, SLAB)])


@jax.jit
def _sc_aggregate(h2, key2, w2):
    h2 = h2
    mesh = plsc.VectorSubcoreMesh(core_axis_name="c", subcore_axis_name="s")
    return pl.kernel(
        _sc_agg_body,
        out_type=jax.ShapeDtypeStruct((NC, ACC_ROWS, FW), jnp.float32),
        mesh=mesh,
        compiler_params=pltpu.CompilerParams(use_tc_tiling_on_sc=False),
        scratch_types=[
            pltpu.VMEM((NCH, C), jnp.int32),
            pltpu.VMEM((NCH, C), jnp.float32),
            pltpu.VMEM((C, 128), jnp.float32),
            pltpu.VMEM((C, 128), jnp.float32),
            pltpu.VMEM((C, 128), jnp.float32),
            pltpu.VMEM((C, 128), jnp.float32),
            pltpu.VMEM((C, FW), jnp.float32),
            pltpu.VMEM((C, FW), jnp.float32),
            pltpu.VMEM((C, FW), jnp.float32),
            pltpu.VMEM((C, FW), jnp.float32),
            pltpu.VMEM((NBUF, C), jnp.int32),
            pltpu.VMEM((NBUF, C), jnp.int32),
            pltpu.VMEM_SHARED((ACC_ROWS, FW), jnp.float32),
            pltpu.SemaphoreType.DMA,
            pltpu.SemaphoreType.DMA,
            pltpu.SemaphoreType.DMA,
            pltpu.SemaphoreType.DMA,
            pltpu.SemaphoreType.DMA,
            pltpu.SemaphoreType.DMA,
            pltpu.SemaphoreType.DMA,
            pltpu.SemaphoreType.DMA,
        ],
    )(h2, key2, w2)


def _sc_degree_body(dst_hbm, deg_hbm, dst_v, ones_v, zb16, dacc_sh):
    cid = lax.axis_index("c")
    sid = lax.axis_index("s")

    ones16 = jnp.ones((16,), jnp.float32)
    zero16 = jnp.zeros((16,), jnp.float32)
    for r in range(C):
        ones_v[r, :] = ones16
        zb16[r, :] = zero16

    base = sid * SLAB

    @pl.when(cid == 0)
    def _():
        pltpu.sync_copy(dst_hbm.at[sid], dst_v)
        for j in range(SLAB // C):
            pltpu.sync_copy(zb16, dacc_sh.at[pl.ds(base + j * C, C)])
        pltpu.sync_copy(zb16.at[pl.ds(0, SLAB % C)],
                        dacc_sh.at[pl.ds(base + (SLAB // C) * C, SLAB % C)])

    plsc.subcore_barrier()

    @pl.when(cid == 0)
    def _():
        def chunk(c, _):
            pltpu.sync_copy(ones_v, dacc_sh.at[dst_v.at[c]], add=True)
            return ()

        lax.fori_loop(0, NCH, chunk, (), unroll=False)

    plsc.subcore_barrier()

    @pl.when(cid == 0)
    def _():
        pltpu.sync_copy(dacc_sh.at[pl.ds(base, SLAB)],
                        deg_hbm.at[pl.ds(base, SLAB)])


@jax.jit
def _sc_degree(dst2):
    mesh = plsc.VectorSubcoreMesh(core_axis_name="c", subcore_axis_name="s")
    return pl.kernel(
        _sc_degree_body,
        out_type=jax.ShapeDtypeStruct((ACC_ROWS, 16), jnp.float32),
        mesh=mesh,
        compiler_params=pltpu.CompilerParams(use_tc_tiling_on_sc=False),
        scratch_types=[
            pltpu.VMEM((NCH, C), jnp.int32),
            pltpu.VMEM((C, 16), jnp.float32),
            pltpu.VMEM((C, 16), jnp.float32),
            pltpu.VMEM_SHARED((ACC_ROWS, 16), jnp.float32),
        ],
    )(dst2)


def _tc_combine_body(relu, h_ref, a_ref, d_ref, ws_ref, b_ref,
                     wnl_ref, wnr_ref, o_ref):
    inv = 1.0 / jnp.maximum(d_ref[:, 0:1], 1.0)
    hp = lax.Precision.HIGHEST
    out = (jnp.dot(h_ref[...], ws_ref[...],
                   preferred_element_type=jnp.float32, precision=hp)
           + b_ref[...]
           + jnp.dot(a_ref[0] * inv, wnl_ref[...],
                     preferred_element_type=jnp.float32, precision=hp)
           + jnp.dot(a_ref[1] * inv, wnr_ref[...],
                     preferred_element_type=jnp.float32, precision=hp))
    if relu:
        out = jnp.maximum(out, 0.0)
    o_ref[...] = out


@functools.partial(jax.jit, static_argnames=("relu",))
def _tc_combine(h, agg, deg, w_self, b_self, w_neigh, relu):
    B = 1000
    grid = (N // B,)
    return pl.pallas_call(
        functools.partial(_tc_combine_body, relu),
        grid=grid,
        in_specs=[
            pl.BlockSpec((B, 128), lambda i: (i, 0)),
            pl.BlockSpec((NC, B, FW), lambda i: (0, i, 0)),
            pl.BlockSpec((B, 16), lambda i: (i, 0)),
            pl.BlockSpec((128, 128), lambda i: (0, 0)),
            pl.BlockSpec((1, 128), lambda i: (0, 0)),
            pl.BlockSpec((FW, 128), lambda i: (0, 0)),
            pl.BlockSpec((FW, 128), lambda i: (0, 0)),
        ],
        out_specs=pl.BlockSpec((B, 128), lambda i: (i, 0)),
        out_shape=jax.ShapeDtypeStruct((N, 128), jnp.float32),
    )(h, agg, deg, w_self, b_self.reshape(1, 128),
      w_neigh[:FW], w_neigh[FW:])


def kernel(in_feat, edge_index, edge_weights,
           W_self0, b_self0, W_neigh0,
           W_self1, b_self1, W_neigh1,
           W_self2, b_self2, W_neigh2):
    src = edge_index[0]
    dst = edge_index[1]
    pad = E_PAD - E
    src_p = jnp.concatenate([src, jnp.zeros((pad,), jnp.int32)])
    dst_agg = jnp.concatenate([dst, jnp.zeros((pad,), jnp.int32)])
    dst_deg = jnp.concatenate([dst, jnp.full((pad,), N, jnp.int32)])
    key2 = ((src_p << KBITS) | dst_agg).reshape(NS, NCH, C)
    dst2 = dst_deg.reshape(NS, NCH, C)
    w2 = jnp.concatenate([edge_weights,
                          jnp.zeros((pad,), jnp.float32)]).reshape(NS, NCH, C)

    deg = _sc_degree(dst2)

    params = [
        (W_self0, b_self0, W_neigh0),
        (W_self1, b_self1, W_neigh1),
        (W_self2, b_self2, W_neigh2),
    ]
    h = in_feat
    for l in range(3):
        ws, bs, wn = params[l]
        agg = _sc_aggregate(h, key2, w2)
        h = _tc_combine(h, agg, deg, ws, bs, wn, relu=(l < 2))
    return h


# speedup vs baseline: 2.8441x; 2.8441x over previous
import functools

import jax
import jax.numpy as jnp
from jax import lax
from jax.experimental import pallas as pl
from jax.experimental.pallas import tpu as pltpu
from jax.experimental.pallas import tpu_sc as plsc

N = 10000
E = 320000

NC = 2
NS = 16
FW = 64
C = 64
NCH = 320
E_PAD = NS * NCH * C
SLAB = 632
ACC_ROWS = NS * SLAB
KBITS = 14


def _splat_lane(v16, i):
    return lax.gather(
        v16,
        jnp.full((16, 1), i, jnp.int32),
        lax.GatherDimensionNumbers(
            offset_dims=(), collapsed_slice_dims=(0,), start_index_map=(0,)),
        (1,),
        mode=lax.GatherScatterMode.PROMISE_IN_BOUNDS,
    )


NBUF = 4


def _sc_agg_body(h2_hbm, key_hbm, w_hbm,
                 out_hbm,
                 key_v, w_v, rbuf0, rbuf1, rbuf2, rbuf3,
                 sbuf0, sbuf1, sbuf2, sbuf3, src_sc, dst_sc,
                 acc_sh, gsem0, gsem1, gsem2, gsem3,
                 ssem0, ssem1, ssem2, ssem3):
    cid = lax.axis_index("c")
    sid = lax.axis_index("s")

    pltpu.sync_copy(key_hbm.at[sid], key_v)
    pltpu.sync_copy(w_hbm.at[sid], w_v)

    rbufs = (rbuf0, rbuf1, rbuf2, rbuf3)
    sbufs = (sbuf0, sbuf1, sbuf2, sbuf3)
    gsems = (gsem0, gsem1, gsem2, gsem3)
    ssems = (ssem0, ssem1, ssem2, ssem3)

    shift16 = jnp.full((16,), KBITS, jnp.int32)
    mask16 = jnp.full((16,), (1 << KBITS) - 1, jnp.int32)

    def unpack_src(row, b):
        for g in range(C // 16):
            sl = pl.ds(g * 16, 16)
            src_sc[b, sl] = lax.shift_right_logical(key_v[row, sl], shift16)

    def unpack_dst(row, b):
        for g in range(C // 16):
            sl = pl.ds(g * 16, 16)
            dst_sc[b, sl] = lax.bitwise_and(key_v[row, sl], mask16)

    zero16 = jnp.zeros((16,), jnp.float32)
    for r in range(C):
        for k in range(FW // 16):
            for sb in sbufs:
                sb[r, pl.ds(k * 16, 16)] = zero16
    base = sid * SLAB
    for j in range(SLAB // C):
        pltpu.sync_copy(sbuf0, acc_sh.at[pl.ds(base + j * C, C)])
    pltpu.sync_copy(sbuf0.at[pl.ds(0, SLAB % C)],
                    acc_sh.at[pl.ds(base + (SLAB // C) * C, SLAB % C)])

    plsc.subcore_barrier()

    htab = h2_hbm.at[cid]

    def scatter_start(b):
        pltpu.async_copy(sbufs[b], acc_sh.at[dst_sc.at[b]], ssems[b], add=True)

    def scatter_wait(b):
        pltpu.make_async_copy(
            sbufs[b], acc_sh.at[dst_sc.at[b]], ssems[b]).wait()

    for b in range(NBUF):
        unpack_src(b, b)
        pltpu.async_copy(htab.at[src_sc.at[b]], rbufs[b], gsems[b])
        unpack_dst(b, b)
        scatter_start(b)

    def chunk_group(t, _):
        c0 = t * NBUF
        for b in range(NBUF):
            cur = c0 + b
            rbuf = rbufs[b]
            sbuf = sbufs[b]
            pltpu.make_async_copy(htab.at[src_sc.at[b]], rbuf, gsems[b]).wait()
            scatter_wait(b)

            unpack_dst(cur, b)

            for g in range(C // 16):
                w16 = w_v[cur, pl.ds(g * 16, 16)]
                for i in range(16):
                    e = g * 16 + i
                    ws = _splat_lane(w16, i)
                    for j in range(FW // 32):
                        p32 = rbuf[e, pl.ds(j * 32, 32)]
                        va, vb = plsc.unpack(
                            p32, format=plsc.PackFormat.INTERLEAVED)
                        sbuf[e, pl.ds(j * 32, 16)] = va * ws
                        sbuf[e, pl.ds(j * 32 + 16, 16)] = vb * ws

            scatter_start(b)

            nxt = cur + NBUF

            @pl.when(nxt < NCH)
            def _():
                unpack_src(nxt, b)
                pltpu.async_copy(htab.at[src_sc.at[b]], rbuf, gsems[b])

        return ()

    lax.fori_loop(0, NCH // NBUF, chunk_group, (), unroll=False)

    for b in range(NBUF):
        scatter_wait(b)

    plsc.subcore_barrier()

    pltpu.sync_copy(acc_sh.at[pl.ds(base, SLAB)],
                    out_hbm.at[cid].at[pl.ds(base, SLAB)])


@jax.jit
def _sc_aggregate(h2, key2, w2):
    mesh = plsc.VectorSubcoreMesh(core_axis_name="c", subcore_axis_name="s")
    return pl.kernel(
        _sc_agg_body,
        out_type=jax.ShapeDtypeStruct((NC, ACC_ROWS, FW), jnp.float32),
        mesh=mesh,
        compiler_params=pltpu.CompilerParams(use_tc_tiling_on_sc=False, needs_layout_passes=False),
        scratch_types=[
            pltpu.VMEM((NCH, C), jnp.int32),
            pltpu.VMEM((NCH, C), jnp.float32),
            pltpu.VMEM((C, FW), jnp.bfloat16),
            pltpu.VMEM((C, FW), jnp.bfloat16),
            pltpu.VMEM((C, FW), jnp.bfloat16),
            pltpu.VMEM((C, FW), jnp.bfloat16),
            pltpu.VMEM((C, FW), jnp.float32),
            pltpu.VMEM((C, FW), jnp.float32),
            pltpu.VMEM((C, FW), jnp.float32),
            pltpu.VMEM((C, FW), jnp.float32),
            pltpu.VMEM((NBUF, C), jnp.int32),
            pltpu.VMEM((NBUF, C), jnp.int32),
            pltpu.VMEM_SHARED((ACC_ROWS, FW), jnp.float32),
            pltpu.SemaphoreType.DMA,
            pltpu.SemaphoreType.DMA,
            pltpu.SemaphoreType.DMA,
            pltpu.SemaphoreType.DMA,
            pltpu.SemaphoreType.DMA,
            pltpu.SemaphoreType.DMA,
            pltpu.SemaphoreType.DMA,
            pltpu.SemaphoreType.DMA,
        ],
    )(h2, key2, w2)


def _sc_degree_body(dst_hbm, deg_hbm, dst_v, ones_v, zb16, dacc_sh):
    cid = lax.axis_index("c")
    sid = lax.axis_index("s")

    ones16 = jnp.ones((16,), jnp.float32)
    zero16 = jnp.zeros((16,), jnp.float32)
    for r in range(C):
        ones_v[r, :] = ones16
        zb16[r, :] = zero16

    base = sid * SLAB

    @pl.when(cid == 0)
    def _():
        pltpu.sync_copy(dst_hbm.at[sid], dst_v)
        for j in range(SLAB // C):
            pltpu.sync_copy(zb16, dacc_sh.at[pl.ds(base + j * C, C)])
        pltpu.sync_copy(zb16.at[pl.ds(0, SLAB % C)],
                        dacc_sh.at[pl.ds(base + (SLAB // C) * C, SLAB % C)])

    plsc.subcore_barrier()

    @pl.when(cid == 0)
    def _():
        def chunk(c, _):
            pltpu.sync_copy(ones_v, dacc_sh.at[dst_v.at[c]], add=True)
            return ()

        lax.fori_loop(0, NCH, chunk, (), unroll=False)

    plsc.subcore_barrier()

    @pl.when(cid == 0)
    def _():
        pltpu.sync_copy(dacc_sh.at[pl.ds(base, SLAB)],
                        deg_hbm.at[pl.ds(base, SLAB)])


@jax.jit
def _sc_degree(dst2):
    mesh = plsc.VectorSubcoreMesh(core_axis_name="c", subcore_axis_name="s")
    return pl.kernel(
        _sc_degree_body,
        out_type=jax.ShapeDtypeStruct((ACC_ROWS, 16), jnp.float32),
        mesh=mesh,
        compiler_params=pltpu.CompilerParams(use_tc_tiling_on_sc=False),
        scratch_types=[
            pltpu.VMEM((NCH, C), jnp.int32),
            pltpu.VMEM((C, 16), jnp.float32),
            pltpu.VMEM((C, 16), jnp.float32),
            pltpu.VMEM_SHARED((ACC_ROWS, 16), jnp.float32),
        ],
    )(dst2)


def _tc_combine_body(relu, h_ref, a_ref, d_ref, ws_ref, b_ref,
                     wnl_ref, wnr_ref, o_ref):
    inv = 1.0 / jnp.maximum(d_ref[:, 0:1], 1.0)
    hp = lax.Precision.HIGHEST
    out = (jnp.dot(h_ref[...], ws_ref[...],
                   preferred_element_type=jnp.float32, precision=hp)
           + b_ref[...]
           + jnp.dot(a_ref[0] * inv, wnl_ref[...],
                     preferred_element_type=jnp.float32, precision=hp)
           + jnp.dot(a_ref[1] * inv, wnr_ref[...],
                     preferred_element_type=jnp.float32, precision=hp))
    if relu:
        out = jnp.maximum(out, 0.0)
    o_ref[...] = out


@functools.partial(jax.jit, static_argnames=("relu",))
def _tc_combine(h, agg, deg, w_self, b_self, w_neigh, relu):
    B = 1000
    grid = (N // B,)
    return pl.pallas_call(
        functools.partial(_tc_combine_body, relu),
        grid=grid,
        in_specs=[
            pl.BlockSpec((B, 128), lambda i: (i, 0)),
            pl.BlockSpec((NC, B, FW), lambda i: (0, i, 0)),
            pl.BlockSpec((B, 16), lambda i: (i, 0)),
            pl.BlockSpec((128, 128), lambda i: (0, 0)),
            pl.BlockSpec((1, 128), lambda i: (0, 0)),
            pl.BlockSpec((FW, 128), lambda i: (0, 0)),
            pl.BlockSpec((FW, 128), lambda i: (0, 0)),
        ],
        out_specs=pl.BlockSpec((B, 128), lambda i: (i, 0)),
        out_shape=jax.ShapeDtypeStruct((N, 128), jnp.float32),
    )(h, agg, deg, w_self, b_self.reshape(1, 128),
      w_neigh[:FW], w_neigh[FW:])


def kernel(in_feat, edge_index, edge_weights,
           W_self0, b_self0, W_neigh0,
           W_self1, b_self1, W_neigh1,
           W_self2, b_self2, W_neigh2):
    src = edge_index[0]
    dst = edge_index[1]
    pad = E_PAD - E
    src_p = jnp.concatenate([src, jnp.zeros((pad,), jnp.int32)])
    dst_agg = jnp.concatenate([dst, jnp.zeros((pad,), jnp.int32)])
    dst_deg = jnp.concatenate([dst, jnp.full((pad,), N, jnp.int32)])
    key2 = ((src_p << KBITS) | dst_agg).reshape(NS, NCH, C)
    dst2 = dst_deg.reshape(NS, NCH, C)
    w2 = jnp.concatenate([edge_weights,
                          jnp.zeros((pad,), jnp.float32)]).reshape(NS, NCH, C)

    deg = _sc_degree(dst2)

    params = [
        (W_self0, b_self0, W_neigh0),
        (W_self1, b_self1, W_neigh1),
        (W_self2, b_self2, W_neigh2),
    ]
    def interleave_bf16(hh):
        g = hh.reshape(N, FW // 32, 2, 16)
        return g.transpose(0, 1, 3, 2).reshape(N, FW).astype(jnp.bfloat16)

    h = in_feat
    for l in range(3):
        ws, bs, wn = params[l]
        h2 = jnp.stack([interleave_bf16(h[:, :FW]),
                        interleave_bf16(h[:, FW:])])
        agg = _sc_aggregate(h2, key2, w2)
        h = _tc_combine(h, agg, deg, ws, bs, wn, relu=(l < 2))
    return h
